# baseline (speedup 1.0000x reference)
import functools

import jax
import jax.numpy as jnp
from jax import lax
from jax.experimental import pallas as pl
from jax.experimental.pallas import tpu as pltpu
from jax.experimental.pallas import tpu_sc as plsc

_NC = 2
_NS = 16
_L = 16
_NW = _NC * _NS


def _softplus(t):
    e = jnp.exp(-jnp.abs(t))
    z = e / (e + 2.0)
    u = z * z
    p = 1.0 / 11.0
    p = p * u + 1.0 / 9.0
    p = p * u + 1.0 / 7.0
    p = p * u + 1.0 / 5.0
    p = p * u + 1.0 / 3.0
    p = p * u + 1.0
    return jnp.maximum(t, 0.0) + 2.0 * z * p


@functools.lru_cache(maxsize=None)
def _build_cw(B, H):
    BW = B // _NW
    mesh = plsc.VectorSubcoreMesh(core_axis_name="c", subcore_axis_name="s")

    @functools.partial(
        pl.kernel,
        out_type=jax.ShapeDtypeStruct((H, B), jnp.float32),
        mesh=mesh,
        compiler_params=pltpu.CompilerParams(
            needs_layout_passes=False, use_tc_tiling_on_sc=False),
        scratch_types=[
            pltpu.VMEM((BW,), jnp.int32),
            pltpu.VMEM((H, BW), jnp.float32),
            pltpu.SemaphoreType.DMA,
        ],
    )
    def cw_kernel(center_hbm, in_t_hbm, cwt_hbm, idx_v, cwt_v, sem):
        wid = lax.axis_index("s") * _NC + lax.axis_index("c")
        b0 = wid * BW
        pltpu.sync_copy(center_hbm.at[pl.ds(b0, BW)], idx_v)
        cps = [pltpu.async_copy(in_t_hbm.at[h].at[idx_v], cwt_v.at[h], sem)
               for h in range(H)]
        for cp in cps:
            cp.wait()
        pltpu.sync_copy(cwt_v, cwt_hbm.at[:, pl.ds(b0, BW)])

    return cw_kernel


@functools.lru_cache(maxsize=None)
def _build_main(B, P, N, H):
    BW = B // _NW
    NG = BW // _L
    assert BW % _L == 0

    mesh = plsc.VectorSubcoreMesh(core_axis_name="c", subcore_axis_name="s")

    @functools.partial(
        pl.kernel,
        out_type=jax.ShapeDtypeStruct((B,), jnp.float32),
        mesh=mesh,
        compiler_params=pltpu.CompilerParams(
            needs_layout_passes=False, use_tc_tiling_on_sc=False),
        scratch_types=[
            pltpu.VMEM((P, BW), jnp.int32),
            pltpu.VMEM((N, BW), jnp.int32),
            pltpu.VMEM((H, BW), jnp.float32),
            pltpu.VMEM((P * BW, H), jnp.float32),
            pltpu.VMEM((N * BW, H), jnp.float32),
            pltpu.VMEM((BW,), jnp.float32),
            pltpu.SemaphoreType.DMA,
        ],
    )
    def sc_kernel(pos_t_hbm, neg_t_hbm, cwt_hbm, out_emb_hbm, res_hbm,
                  pos_idx, neg_idx, cwt_v, pos_rows, neg_rows, res_v, sem):
        wid = lax.axis_index("s") * _NC + lax.axis_index("c")
        b0 = wid * BW

        pltpu.sync_copy(cwt_hbm.at[:, pl.ds(b0, BW)], cwt_v)
        for j in range(P):
            pltpu.sync_copy(pos_t_hbm.at[j, pl.ds(b0, BW)], pos_idx.at[j])
        for j in range(N):
            pltpu.sync_copy(neg_t_hbm.at[j, pl.ds(b0, BW)], neg_idx.at[j])

        copies = []
        for j in range(P):
            copies.append(pltpu.async_copy(
                out_emb_hbm.at[pos_idx.at[j]],
                pos_rows.at[pl.ds(j * BW, BW)], sem))
        for j in range(N):
            copies.append(pltpu.async_copy(
                out_emb_hbm.at[neg_idx.at[j]],
                neg_rows.at[pl.ds(j * BW, BW)], sem))
        for cp in copies:
            cp.wait()

        iota = lax.iota(jnp.int32, _L)
        cols = [jnp.full((_L,), h, jnp.int32) for h in range(H)]

        def group(g, _):
            base = g * _L
            lane = base + iota
            cwv = [cwt_v[h, pl.ds(pl.multiple_of(base, _L), _L)]
                   for h in range(H)]

            def pos_body(j, tot):
                r = j * BW + lane
                s = plsc.load_gather(pos_rows, [r, cols[0]]) * cwv[0]
                for h in range(1, H):
                    s = s + plsc.load_gather(pos_rows, [r, cols[h]]) * cwv[h]
                return tot + _softplus(-s)

            def neg_body(j, tot):
                r = j * BW + lane
                s = plsc.load_gather(neg_rows, [r, cols[0]]) * cwv[0]
                for h in range(1, H):
                    s = s + plsc.load_gather(neg_rows, [r, cols[h]]) * cwv[h]
                return tot + _softplus(s)

            tot = lax.fori_loop(0, P, pos_body, jnp.zeros((_L,), jnp.float32))
            tot = lax.fori_loop(0, N, neg_body, tot)
            res_v[pl.ds(pl.multiple_of(base, _L), _L)] = tot
            return 0

        lax.fori_loop(0, NG, group, 0)
        pltpu.sync_copy(res_v, res_hbm.at[pl.ds(b0, BW)])

    return sc_kernel


def kernel(center_word_idx, pos_words_idx, neg_words_idx, in_emb, out_emb):
    B, = center_word_idx.shape
    P = pos_words_idx.shape[1]
    N = neg_words_idx.shape[1]
    H = in_emb.shape[1]
    cwt = _build_cw(B, H)(center_word_idx.astype(jnp.int32), in_emb.T)
    return _build_main(B, P, N, H)(
        pos_words_idx.T.astype(jnp.int32),
        neg_words_idx.T.astype(jnp.int32),
        cwt, out_emb)


# speedup vs baseline: 3.1238x; 3.1238x over previous
import functools

import jax
import jax.numpy as jnp
from jax import lax
from jax.experimental import pallas as pl
from jax.experimental.pallas import tpu as pltpu
from jax.experimental.pallas import tpu_sc as plsc

_NC = 2
_NS = 16
_L = 16
_NW = _NC * _NS


def _softplus(t):
    e = jnp.exp(-jnp.abs(t))
    z = e / (e + 2.0)
    u = z * z
    p = 1.0 / 11.0
    p = p * u + 1.0 / 9.0
    p = p * u + 1.0 / 7.0
    p = p * u + 1.0 / 5.0
    p = p * u + 1.0 / 3.0
    p = p * u + 1.0
    return jnp.maximum(t, 0.0) + 2.0 * z * p


@functools.lru_cache(maxsize=None)
def _build(B, P, N, H):
    BW = B // _NW
    NG = BW // _L
    assert BW % _L == 0

    mesh = plsc.VectorSubcoreMesh(core_axis_name="c", subcore_axis_name="s")

    @functools.partial(
        pl.kernel,
        out_type=jax.ShapeDtypeStruct((B,), jnp.float32),
        mesh=mesh,
        compiler_params=pltpu.CompilerParams(
            needs_layout_passes=False, use_tc_tiling_on_sc=False),
        scratch_types=[
            pltpu.VMEM((BW,), jnp.int32),
            pltpu.VMEM((P, BW), jnp.int32),
            pltpu.VMEM((N, BW), jnp.int32),
            pltpu.VMEM((BW, H), jnp.float32),
            pltpu.VMEM((P * BW, H), jnp.float32),
            pltpu.VMEM((N * BW, H), jnp.float32),
            pltpu.VMEM((BW,), jnp.float32),
            pltpu.SemaphoreType.DMA,
        ],
    )
    def sc_kernel(center_hbm, pos_t_hbm, neg_t_hbm, in_emb_hbm, out_emb_hbm,
                  res_hbm, cw_idx, pos_idx, neg_idx, cw_rows, pos_rows,
                  neg_rows, res_v, sem):
        wid = lax.axis_index("s") * _NC + lax.axis_index("c")
        b0 = wid * BW

        pltpu.sync_copy(center_hbm.at[pl.ds(b0, BW)], cw_idx)
        for j in range(P):
            pltpu.sync_copy(pos_t_hbm.at[j, pl.ds(b0, BW)], pos_idx.at[j])
        for j in range(N):
            pltpu.sync_copy(neg_t_hbm.at[j, pl.ds(b0, BW)], neg_idx.at[j])

        copies = [pltpu.async_copy(in_emb_hbm.at[cw_idx], cw_rows, sem)]
        for j in range(P):
            copies.append(pltpu.async_copy(
                out_emb_hbm.at[pos_idx.at[j]],
                pos_rows.at[pl.ds(j * BW, BW)], sem))
        for j in range(N):
            copies.append(pltpu.async_copy(
                out_emb_hbm.at[neg_idx.at[j]],
                neg_rows.at[pl.ds(j * BW, BW)], sem))
        for cp in copies:
            cp.wait()

        iota = lax.iota(jnp.int32, _L)
        cols = [jnp.full((_L,), h, jnp.int32) for h in range(H)]

        def group(g, _):
            lane = g * _L + iota
            cwv = [plsc.load_gather(cw_rows, [lane, cols[h]])
                   for h in range(H)]

            def pos_body(j, tot):
                r = j * BW + lane
                s = plsc.load_gather(pos_rows, [r, cols[0]]) * cwv[0]
                for h in range(1, H):
                    s = s + plsc.load_gather(pos_rows, [r, cols[h]]) * cwv[h]
                return tot + _softplus(-s)

            def neg_body(j, tot):
                r = j * BW + lane
                s = plsc.load_gather(neg_rows, [r, cols[0]]) * cwv[0]
                for h in range(1, H):
                    s = s + plsc.load_gather(neg_rows, [r, cols[h]]) * cwv[h]
                return tot + _softplus(s)

            tot = lax.fori_loop(0, P, pos_body, jnp.zeros((_L,), jnp.float32))
            tot = lax.fori_loop(0, N, neg_body, tot)
            res_v[pl.ds(pl.multiple_of(g * _L, _L), _L)] = tot
            return 0

        lax.fori_loop(0, NG, group, 0)
        pltpu.sync_copy(res_v, res_hbm.at[pl.ds(b0, BW)])

    return sc_kernel


def kernel(center_word_idx, pos_words_idx, neg_words_idx, in_emb, out_emb):
    B, = center_word_idx.shape
    P = pos_words_idx.shape[1]
    N = neg_words_idx.shape[1]
    H = in_emb.shape[1]
    fn = _build(B, P, N, H)
    return fn(center_word_idx.astype(jnp.int32),
              pos_words_idx.T.astype(jnp.int32),
              neg_words_idx.T.astype(jnp.int32),
              in_emb, out_emb)


# speedup vs baseline: 3.1275x; 1.0012x over previous
import functools

import jax
import jax.numpy as jnp
from jax import lax
from jax.experimental import pallas as pl
from jax.experimental.pallas import tpu as pltpu
from jax.experimental.pallas import tpu_sc as plsc

_NC = 2
_NS = 16
_L = 16
_NW = _NC * _NS


def _softplus(t):
    e = jnp.exp(-jnp.abs(t))
    z = e / (e + 2.0)
    u = z * z
    p = 1.0 / 11.0
    p = p * u + 1.0 / 9.0
    p = p * u + 1.0 / 7.0
    p = p * u + 1.0 / 5.0
    p = p * u + 1.0 / 3.0
    p = p * u + 1.0
    return jnp.maximum(t, 0.0) + 2.0 * z * p


@functools.lru_cache(maxsize=None)
def _build(B, P, N, H):
    BW = B // _NW
    NG = BW // _L
    assert BW % _L == 0

    mesh = plsc.VectorSubcoreMesh(core_axis_name="c", subcore_axis_name="s")

    @functools.partial(
        pl.kernel,
        out_type=jax.ShapeDtypeStruct((B,), jnp.float32),
        mesh=mesh,
        compiler_params=pltpu.CompilerParams(
            needs_layout_passes=False, use_tc_tiling_on_sc=False),
        scratch_types=[
            pltpu.VMEM((BW,), jnp.int32),
            pltpu.VMEM((P, BW), jnp.int32),
            pltpu.VMEM((N, BW), jnp.int32),
            pltpu.VMEM((BW, H), jnp.float32),
            pltpu.VMEM((P * BW, H), jnp.float32),
            pltpu.VMEM((N * BW, H), jnp.float32),
            pltpu.VMEM((BW,), jnp.float32),
            pltpu.SemaphoreType.DMA,
        ],
    )
    def sc_kernel(center_hbm, pos_t_hbm, neg_t_hbm, in_emb_hbm, out_emb_hbm,
                  res_hbm, cw_idx, pos_idx, neg_idx, cw_rows, pos_rows,
                  neg_rows, res_v, sem):
        wid = lax.axis_index("s") * _NC + lax.axis_index("c")
        b0 = wid * BW

        pltpu.sync_copy(center_hbm.at[pl.ds(b0, BW)], cw_idx)
        for j in range(P):
            pltpu.sync_copy(pos_t_hbm.at[j, pl.ds(b0, BW)], pos_idx.at[j])
        for j in range(N):
            pltpu.sync_copy(neg_t_hbm.at[j, pl.ds(b0, BW)], neg_idx.at[j])

        copies = [pltpu.async_copy(in_emb_hbm.at[cw_idx], cw_rows, sem)]
        for j in range(P):
            copies.append(pltpu.async_copy(
                out_emb_hbm.at[pos_idx.at[j]],
                pos_rows.at[pl.ds(j * BW, BW)], sem))
        for j in range(N):
            copies.append(pltpu.async_copy(
                out_emb_hbm.at[neg_idx.at[j]],
                neg_rows.at[pl.ds(j * BW, BW)], sem))
        for cp in copies:
            cp.wait()

        iota = lax.iota(jnp.int32, _L)
        cols = [jnp.full((_L,), h, jnp.int32) for h in range(H)]

        def group(g, _):
            lane = g * _L + iota
            cwv = [plsc.load_gather(cw_rows, [lane, cols[h]])
                   for h in range(H)]

            def pos_body(j, tot):
                r = j * BW + lane
                s = plsc.load_gather(pos_rows, [r, cols[0]]) * cwv[0]
                for h in range(1, H):
                    s = s + plsc.load_gather(pos_rows, [r, cols[h]]) * cwv[h]
                return tot + _softplus(-s)

            def neg_body(j, tot):
                r = j * BW + lane
                s = plsc.load_gather(neg_rows, [r, cols[0]]) * cwv[0]
                for h in range(1, H):
                    s = s + plsc.load_gather(neg_rows, [r, cols[h]]) * cwv[h]
                return tot + _softplus(s)

            tot = lax.fori_loop(0, P, pos_body, jnp.zeros((_L,), jnp.float32))
            tot = lax.fori_loop(0, N, neg_body, tot)
            res_v[pl.ds(pl.multiple_of(g * _L, _L), _L)] = tot
            return 0

        lax.fori_loop(0, NG, group, 0)
        pltpu.sync_copy(res_v, res_hbm.at[pl.ds(b0, BW)])

    return sc_kernel


def kernel(center_word_idx, pos_words_idx, neg_words_idx, in_emb, out_emb):
    B, = center_word_idx.shape
    P = pos_words_idx.shape[1]
    N = neg_words_idx.shape[1]
    H = in_emb.shape[1]
    fn = _build(B, P, N, H)
    V = in_emb.shape[0]
    in_lin = jnp.reshape(jnp.reshape(in_emb, (V * H,)), (V, H))
    out_lin = jnp.reshape(jnp.reshape(out_emb, (V * H,)), (V, H))
    return fn(center_word_idx.astype(jnp.int32),
              pos_words_idx.T.astype(jnp.int32),
              neg_words_idx.T.astype(jnp.int32),
              in_lin, out_lin)
